# Initial kernel scaffold; baseline (speedup 1.0000x reference)
import jax
import jax.numpy as jnp
from jax.experimental import pallas as pl


def kernel(x, edge_index, norm, y, W_conv, b_conv, Wc1, bc1, Wc2, bc2, bn_gamma, bn_beta):
    raise NotImplementedError("write your pallas kernel here")


# speedup vs baseline: 3.6805x; 3.6805x over previous
import functools

import jax
import jax.numpy as jnp
from jax import lax
from jax.experimental import pallas as pl
from jax.experimental.pallas import tpu as pltpu
from jax.experimental.pallas import tpu_sc as plsc

N_NODES = 10000
N_HE = 10000
N_EDGES = 320000
D = 128
EPS = 1e-5

NC = 2
NS = 16
NW = NC * NS
EPW = N_EDGES // NW
CH = 80
NCHUNK = EPW // CH
ZR = 80
NZCOPY = N_NODES // ZR
CNTP = 10240

BM = 2000
GRID = N_NODES // BM


def _ln(x):
    m = jnp.mean(x, axis=-1, keepdims=True)
    v = jnp.mean((x - m) ** 2, axis=-1, keepdims=True)
    return (x - m) / jnp.sqrt(v + EPS)


def _mlp_relu(x, W1, b1, W2, b2):
    x = _ln(x)
    x = jax.nn.relu(jnp.dot(x, W1, preferred_element_type=jnp.float32) + b1)
    x = _ln(x)
    return jax.nn.relu(jnp.dot(x, W2, preferred_element_type=jnp.float32) + b2)


def _full(shape):
    return pl.BlockSpec(shape, lambda i: tuple(0 for _ in shape))


def _rows(shape):
    return pl.BlockSpec(shape, lambda i: (i,) + tuple(0 for _ in shape[1:]))


def _prep_dst(dst):
    def body(d_ref, o_ref):
        dd = d_ref[...]
        o_ref[...] = dd - jnp.min(dd)

    d2 = dst.reshape(2500, 128)
    out = pl.pallas_call(
        body,
        grid=(1,),
        in_specs=[_full((2500, 128))],
        out_specs=_full((2500, 128)),
        out_shape=jax.ShapeDtypeStruct((2500, 128), jnp.int32),
    )(d2)
    return out.reshape(N_EDGES)


def _enc(x, W1, b1, W2, b2, pre=None, gamma=None, beta=None):
    if pre == "bn":
        def body(x_ref, g_ref, be_ref, W1_ref, b1_ref, W2_ref, b2_ref, o_ref):
            xb = x_ref[...]
            xb = jax.nn.relu(g_ref[...] * (xb / jnp.sqrt(1.0 + EPS)) + be_ref[...])
            o_ref[...] = _mlp_relu(xb, W1_ref[...], b1_ref[...], W2_ref[...], b2_ref[...])

        args = (x, gamma.reshape(1, D), beta.reshape(1, D),
                W1, b1.reshape(1, D), W2, b2.reshape(1, D))
        in_specs = [_rows((BM, D)), _full((1, D)), _full((1, D)),
                    _full((D, D)), _full((1, D)), _full((D, D)), _full((1, D))]
    else:
        def body(x_ref, W1_ref, b1_ref, W2_ref, b2_ref, o_ref):
            o_ref[...] = _mlp_relu(x_ref[...], W1_ref[...], b1_ref[...],
                                   W2_ref[...], b2_ref[...])

        args = (x, W1, b1.reshape(1, D), W2, b2.reshape(1, D))
        in_specs = [_rows((BM, D)), _full((D, D)), _full((1, D)),
                    _full((D, D)), _full((1, D))]

    return pl.pallas_call(
        body,
        grid=(GRID,),
        in_specs=in_specs,
        out_specs=_rows((BM, D)),
        out_shape=jax.ShapeDtypeStruct((N_NODES, D), jnp.float32),
    )(*args)


def _dec(acc, cnt, W1, b1, W2, b2):
    def body(a0_ref, a1_ref, c_ref, W1_ref, b1_ref, W2_ref, b2_ref,
             o_ref, cs_ref, ss_ref):
        i = pl.program_id(0)
        a = a0_ref[...] + a1_ref[...]
        cb = c_ref[...]
        c = cb[:, 0] + cb[:, 1]
        m = a / jnp.maximum(c, 1.0)[:, None]
        h = _mlp_relu(m, W1_ref[...], b1_ref[...], W2_ref[...], b2_ref[...])
        o_ref[...] = h

        @pl.when(i == 0)
        def _():
            cs_ref[...] = jnp.zeros_like(cs_ref)
            ss_ref[...] = jnp.zeros_like(ss_ref)

        cs_ref[...] += jnp.sum(h, axis=0, keepdims=True)
        ss_ref[...] += jnp.sum(h * h).reshape(1, 1)

    return pl.pallas_call(
        body,
        grid=(GRID,),
        in_specs=[_rows((BM, D)), _rows((BM, D)), _rows((BM, 2)),
                  _full((D, D)), _full((1, D)), _full((D, D)), _full((1, D))],
        out_specs=[_rows((BM, D)), _full((1, D)), _full((1, 1))],
        out_shape=[jax.ShapeDtypeStruct((N_NODES, D), jnp.float32),
                   jax.ShapeDtypeStruct((1, D), jnp.float32),
                   jax.ShapeDtypeStruct((1, 1), jnp.float32)],
    )(acc[0], acc[1], cnt.T[:N_NODES], W1, b1.reshape(1, D), W2,
      b2.reshape(1, D))


def _center(x, cs, ss):
    def body(x_ref, cs_ref, ss_ref, o_ref):
        m = cs_ref[...] / N_NODES
        msq = (ss_ref[0, 0] - N_NODES * jnp.sum(m * m)) / N_NODES
        inv = lax.rsqrt(EPS + msq)
        o_ref[...] = (x_ref[...] - m) * inv

    return pl.pallas_call(
        body,
        grid=(GRID,),
        in_specs=[_rows((BM, D)), _full((1, D)), _full((1, 1))],
        out_specs=_rows((BM, D)),
        out_shape=jax.ShapeDtypeStruct((N_NODES, D), jnp.float32),
    )(x, cs, ss)


def _classifier(c1, c5, c9, Wc1, bc1, Wc2, bc2):
    W2p = jnp.pad(Wc2, ((0, 0), (0, D - Wc2.shape[1])))
    b2p = jnp.pad(bc2, (0, D - bc2.shape[0])).reshape(1, D)

    def body(c1_ref, c5_ref, c9_ref, W_ref, b1_ref, W2_ref, b2_ref, o_ref):
        W = W_ref[...]
        h = (jnp.dot(c1_ref[...], W[0:D], preferred_element_type=jnp.float32)
             + jnp.dot(c5_ref[...], W[D:2 * D], preferred_element_type=jnp.float32)
             + jnp.dot(c9_ref[...], W[2 * D:3 * D], preferred_element_type=jnp.float32)
             + b1_ref[...])
        h = _ln(jax.nn.relu(h))
        o_ref[...] = jnp.dot(h, W2_ref[...], preferred_element_type=jnp.float32) + b2_ref[...]

    out = pl.pallas_call(
        body,
        grid=(GRID,),
        in_specs=[_rows((BM, D)), _rows((BM, D)), _rows((BM, D)),
                  _full((3 * D, D)), _full((1, D)), _full((D, D)), _full((1, D))],
        out_specs=_rows((BM, D)),
        out_shape=jax.ShapeDtypeStruct((N_HE, D), jnp.float32),
    )(c1, c5, c9, Wc1, bc1.reshape(1, D), W2p, b2p)
    return out[:, :Wc2.shape[1]]


def _make_prop(with_counts):
    out_type = [jax.ShapeDtypeStruct((NC, N_NODES, D), jnp.float32)]
    scratch = [
        pltpu.VMEM((CH,), jnp.int32),
        pltpu.VMEM((CH,), jnp.int32),
        pltpu.VMEM((CH,), jnp.float32),
        pltpu.VMEM((CH, D), jnp.float32),
        pltpu.VMEM((ZR, D), jnp.float32),
        pltpu.MemorySpace.VMEM_SHARED((N_NODES, D), jnp.float32),
        pltpu.SemaphoreType.DMA,
    ]
    if with_counts:
        out_type.append(jax.ShapeDtypeStruct((NC, CNTP), jnp.float32))
        scratch += [
            pltpu.VMEM((CH,), jnp.float32),
            pltpu.VMEM((CNTP // NS,), jnp.float32),
            pltpu.MemorySpace.VMEM_SHARED((CNTP,), jnp.float32),
        ]

    mesh = plsc.VectorSubcoreMesh(core_axis_name="c", subcore_axis_name="s",
                                  num_cores=NC, num_subcores=NS)

    def body(x_hbm, src_hbm, dst_hbm, nrm_hbm, *refs):
        if with_counts:
            (acc_out, cnt_out, sidx, didx, nrmv, rows, zrow, acc, sem,
             ones, zcnt, cacc) = refs
        else:
            acc_out, sidx, didx, nrmv, rows, zrow, acc, sem = refs
        cid = lax.axis_index("c")
        sid = lax.axis_index("s")
        wid = cid * NS + sid

        @pl.loop(0, ZR)
        def _zr(r):
            for q in range(D // 16):
                zrow[r, pl.ds(q * 16, 16)] = jnp.zeros((16,), jnp.float32)

        for j in range(-(-NZCOPY // NS)):
            t = sid + j * NS

            @pl.when(t < NZCOPY)
            def _():
                pltpu.sync_copy(zrow, acc.at[pl.ds(t * ZR, ZR)])

        if with_counts:
            for q in range(CH // 16):
                ones[pl.ds(q * 16, 16)] = jnp.ones((16,), jnp.float32)
            for q in range((CNTP // NS) // 16):
                zcnt[pl.ds(q * 16, 16)] = jnp.zeros((16,), jnp.float32)
            pltpu.sync_copy(zcnt, cacc.at[pl.ds(sid * (CNTP // NS), CNTP // NS)])

        plsc.subcore_barrier()

        @pl.loop(0, NCHUNK)
        def _chunk(k):
            base = wid * EPW + k * CH
            pltpu.sync_copy(src_hbm.at[pl.ds(base, CH)], sidx)
            pltpu.sync_copy(dst_hbm.at[pl.ds(base, CH)], didx)
            pltpu.sync_copy(nrm_hbm.at[pl.ds(base, CH)], nrmv)
            pltpu.async_copy(x_hbm.at[sidx], rows, sem).wait()
            for g in range(CH // 16):
                nv = nrmv[pl.ds(g * 16, 16)]
                for j in range(16):
                    e = g * 16 + j
                    b = lax.gather(
                        nv, jnp.full((16, 1), j, jnp.int32),
                        lax.GatherDimensionNumbers(
                            offset_dims=(), collapsed_slice_dims=(0,),
                            start_index_map=(0,)),
                        (1,), mode=lax.GatherScatterMode.PROMISE_IN_BOUNDS)
                    for q in range(D // 16):
                        sl = pl.ds(q * 16, 16)
                        rows[e, sl] = rows[e, sl] * b
            pltpu.sync_copy(rows, acc.at[didx], add=True)
            if with_counts:
                pltpu.sync_copy(ones, cacc.at[didx], add=True)

        plsc.subcore_barrier()

        for j in range(-(-NZCOPY // NS)):
            t = sid + j * NS

            @pl.when(t < NZCOPY)
            def _():
                sl = pl.ds(t * ZR, ZR)
                pltpu.sync_copy(acc.at[sl], acc_out.at[cid, sl])
        if with_counts:
            sl = pl.ds(sid * (CNTP // NS), CNTP // NS)
            pltpu.sync_copy(cacc.at[sl], cnt_out.at[cid, sl])

    return pl.kernel(body, out_type=out_type, mesh=mesh, scratch_types=scratch)


_prop_cnt = _make_prop(True)
_prop = _make_prop(False)


def kernel(x, edge_index, norm, y, W_conv, b_conv, Wc1, bc1, Wc2, bc2,
           bn_gamma, bn_beta):
    src = edge_index[0]
    dst = _prep_dst(edge_index[1])

    def half(x_in, gather_i, scatter_i, Wl, bl, cnt=None, pre=None,
             g=None, be=None):
        xe = _enc(x_in, Wl[0], bl[0], Wl[1], bl[1], pre=pre, gamma=g, beta=be)
        if cnt is None:
            acc, cnt = _prop_cnt(xe, gather_i, scatter_i, norm)
        else:
            (acc,) = _prop(xe, gather_i, scatter_i, norm)
        xd, cs, ss = _dec(acc, cnt, Wl[2], bl[2], Wl[3], bl[3])
        return _center(xd, cs, ss), cnt

    c1, cnt_d = half(x, src, dst, W_conv[0], b_conv[0])
    c3, cnt_s = half(c1, dst, src, W_conv[3], b_conv[3],
                     pre="bn", g=bn_gamma[0], be=bn_beta[0])
    c5, _ = half(c3, src, dst, W_conv[1], b_conv[1], cnt=cnt_d,
                 pre="bn", g=bn_gamma[2], be=bn_beta[2])
    c7, _ = half(c5, dst, src, W_conv[4], b_conv[4], cnt=cnt_s,
                 pre="bn", g=bn_gamma[1], be=bn_beta[1])
    c9, _ = half(c7, src, dst, W_conv[2], b_conv[2], cnt=cnt_d,
                 pre="bn", g=bn_gamma[3], be=bn_beta[3])

    edge_score = _classifier(c1, c5, c9, Wc1, bc1, Wc2, bc2)
    return (edge_score, c9, c7)


# speedup vs baseline: 5.0005x; 1.3586x over previous
import jax
import jax.numpy as jnp
from jax import lax
from jax.experimental import pallas as pl
from jax.experimental.pallas import tpu as pltpu
from jax.experimental.pallas import tpu_sc as plsc

N_NODES = 10000
N_HE = 10000
N_EDGES = 320000
D = 128
EPS = 1e-5

NC = 2
NS = 16
NW = NC * NS
EPW = N_EDGES // NW
CH = 80
NCHUNK = EPW // CH
ZR = 80
NZCOPY = N_NODES // ZR
CNTP = 10240

BM = 2000
GRID = N_NODES // BM


def _ln(x):
    m = jnp.mean(x, axis=-1, keepdims=True)
    v = jnp.mean((x - m) ** 2, axis=-1, keepdims=True)
    return (x - m) / jnp.sqrt(v + EPS)


def _mlp_relu(x, W1, b1, W2, b2):
    x = _ln(x)
    x = jax.nn.relu(jnp.dot(x, W1, preferred_element_type=jnp.float32) + b1)
    x = _ln(x)
    return jax.nn.relu(jnp.dot(x, W2, preferred_element_type=jnp.float32) + b2)


def _full(shape):
    return pl.BlockSpec(shape, lambda i: tuple(0 for _ in shape))


def _rows(shape):
    return pl.BlockSpec(shape, lambda i: (i,) + tuple(0 for _ in shape[1:]))


def _prep_dst(dst):
    def body(d_ref, o_ref):
        dd = d_ref[...]
        o_ref[...] = dd - jnp.min(dd)

    d2 = dst.reshape(2500, 128)
    out = pl.pallas_call(
        body,
        grid=(1,),
        in_specs=[_full((2500, 128))],
        out_specs=_full((2500, 128)),
        out_shape=jax.ShapeDtypeStruct((2500, 128), jnp.int32),
    )(d2)
    return out.reshape(N_EDGES)


def _enc(x, W1, b1, W2, b2, pre=None, gamma=None, beta=None):
    if pre == "bn":
        def body(x_ref, g_ref, be_ref, W1_ref, b1_ref, W2_ref, b2_ref, o_ref):
            xb = x_ref[...]
            xb = jax.nn.relu(g_ref[...] * (xb / jnp.sqrt(1.0 + EPS)) + be_ref[...])
            o_ref[...] = _mlp_relu(xb, W1_ref[...], b1_ref[...], W2_ref[...], b2_ref[...])

        args = (x, gamma.reshape(1, D), beta.reshape(1, D),
                W1, b1.reshape(1, D), W2, b2.reshape(1, D))
        in_specs = [_rows((BM, D)), _full((1, D)), _full((1, D)),
                    _full((D, D)), _full((1, D)), _full((D, D)), _full((1, D))]
    else:
        def body(x_ref, W1_ref, b1_ref, W2_ref, b2_ref, o_ref):
            o_ref[...] = _mlp_relu(x_ref[...], W1_ref[...], b1_ref[...],
                                   W2_ref[...], b2_ref[...])

        args = (x, W1, b1.reshape(1, D), W2, b2.reshape(1, D))
        in_specs = [_rows((BM, D)), _full((D, D)), _full((1, D)),
                    _full((D, D)), _full((1, D))]

    return pl.pallas_call(
        body,
        grid=(GRID,),
        in_specs=in_specs,
        out_specs=_rows((BM, D)),
        out_shape=jax.ShapeDtypeStruct((N_NODES, D), jnp.float32),
    )(*args)


def _dec(acc, cnt, W1, b1, W2, b2):
    def body(a0_ref, a1_ref, c_ref, W1_ref, b1_ref, W2_ref, b2_ref,
             o_ref, cs_ref, ss_ref):
        i = pl.program_id(0)
        a = a0_ref[...] + a1_ref[...]
        cb = c_ref[...]
        c = cb[:, 0] + cb[:, 1]
        m = a / jnp.maximum(c, 1.0)[:, None]
        h = _mlp_relu(m, W1_ref[...], b1_ref[...], W2_ref[...], b2_ref[...])
        o_ref[...] = h

        @pl.when(i == 0)
        def _():
            cs_ref[...] = jnp.zeros_like(cs_ref)
            ss_ref[...] = jnp.zeros_like(ss_ref)

        cs_ref[...] += jnp.sum(h, axis=0, keepdims=True)
        ss_ref[...] += jnp.sum(h * h).reshape(1, 1)

    return pl.pallas_call(
        body,
        grid=(GRID,),
        in_specs=[_rows((BM, D)), _rows((BM, D)), _rows((BM, 2)),
                  _full((D, D)), _full((1, D)), _full((D, D)), _full((1, D))],
        out_specs=[_rows((BM, D)), _full((1, D)), _full((1, 1))],
        out_shape=[jax.ShapeDtypeStruct((N_NODES, D), jnp.float32),
                   jax.ShapeDtypeStruct((1, D), jnp.float32),
                   jax.ShapeDtypeStruct((1, 1), jnp.float32)],
    )(acc[0], acc[1], cnt.T[:N_NODES], W1, b1.reshape(1, D), W2,
      b2.reshape(1, D))


def _center(x, cs, ss):
    def body(x_ref, cs_ref, ss_ref, o_ref):
        m = cs_ref[...] / N_NODES
        msq = (ss_ref[0, 0] - N_NODES * jnp.sum(m * m)) / N_NODES
        inv = lax.rsqrt(EPS + msq)
        o_ref[...] = (x_ref[...] - m) * inv

    return pl.pallas_call(
        body,
        grid=(GRID,),
        in_specs=[_rows((BM, D)), _full((1, D)), _full((1, 1))],
        out_specs=_rows((BM, D)),
        out_shape=jax.ShapeDtypeStruct((N_NODES, D), jnp.float32),
    )(x, cs, ss)


def _classifier(c1, c5, c9, Wc1, bc1, Wc2, bc2):
    W2p = jnp.pad(Wc2, ((0, 0), (0, D - Wc2.shape[1])))
    b2p = jnp.pad(bc2, (0, D - bc2.shape[0])).reshape(1, D)

    def body(c1_ref, c5_ref, c9_ref, W_ref, b1_ref, W2_ref, b2_ref, o_ref):
        W = W_ref[...]
        h = (jnp.dot(c1_ref[...], W[0:D], preferred_element_type=jnp.float32)
             + jnp.dot(c5_ref[...], W[D:2 * D], preferred_element_type=jnp.float32)
             + jnp.dot(c9_ref[...], W[2 * D:3 * D], preferred_element_type=jnp.float32)
             + b1_ref[...])
        h = _ln(jax.nn.relu(h))
        o_ref[...] = jnp.dot(h, W2_ref[...], preferred_element_type=jnp.float32) + b2_ref[...]

    out = pl.pallas_call(
        body,
        grid=(GRID,),
        in_specs=[_rows((BM, D)), _rows((BM, D)), _rows((BM, D)),
                  _full((3 * D, D)), _full((1, D)), _full((D, D)), _full((1, D))],
        out_specs=_rows((BM, D)),
        out_shape=jax.ShapeDtypeStruct((N_HE, D), jnp.float32),
    )(c1, c5, c9, Wc1, bc1.reshape(1, D), W2p, b2p)
    return out[:, :Wc2.shape[1]]


def _make_prop(with_counts):
    out_type = [jax.ShapeDtypeStruct((NC, N_NODES, D), jnp.float32)]
    scratch = [
        pltpu.VMEM((CH,), jnp.int32),
        pltpu.VMEM((CH,), jnp.int32),
        pltpu.VMEM((CH,), jnp.int32),
        pltpu.VMEM((CH,), jnp.int32),
        pltpu.VMEM((CH,), jnp.float32),
        pltpu.VMEM((CH,), jnp.float32),
        pltpu.VMEM((CH, D), jnp.float32),
        pltpu.VMEM((CH, D), jnp.float32),
        pltpu.VMEM((ZR, D), jnp.float32),
        pltpu.MemorySpace.VMEM_SHARED((N_NODES, D), jnp.float32),
        pltpu.SemaphoreType.DMA,
        pltpu.SemaphoreType.DMA,
    ]
    if with_counts:
        out_type.append(jax.ShapeDtypeStruct((NC, CNTP), jnp.float32))
        scratch += [
            pltpu.VMEM((CH,), jnp.float32),
            pltpu.VMEM((CNTP // NS,), jnp.float32),
            pltpu.MemorySpace.VMEM_SHARED((CNTP,), jnp.float32),
        ]

    mesh = plsc.VectorSubcoreMesh(core_axis_name="c", subcore_axis_name="s",
                                  num_cores=NC, num_subcores=NS)

    def body(x_hbm, src_hbm, dst_hbm, nrm_hbm, *refs):
        if with_counts:
            (acc_out, cnt_out, sidx0, sidx1, didx0, didx1, nrm0, nrm1,
             rows0, rows1, zrow, acc, sem0, sem1, ones, zcnt, cacc) = refs
        else:
            (acc_out, sidx0, sidx1, didx0, didx1, nrm0, nrm1,
             rows0, rows1, zrow, acc, sem0, sem1) = refs
        bufs = ((sidx0, didx0, nrm0, rows0, sem0),
                (sidx1, didx1, nrm1, rows1, sem1))
        cid = lax.axis_index("c")
        sid = lax.axis_index("s")
        wid = cid * NS + sid

        @pl.loop(0, ZR)
        def _zr(r):
            for q in range(D // 16):
                zrow[r, pl.ds(q * 16, 16)] = jnp.zeros((16,), jnp.float32)

        for j in range(-(-NZCOPY // NS)):
            t = sid + j * NS

            @pl.when(t < NZCOPY)
            def _():
                pltpu.sync_copy(zrow, acc.at[pl.ds(t * ZR, ZR)])

        if with_counts:
            for q in range(CH // 16):
                ones[pl.ds(q * 16, 16)] = jnp.ones((16,), jnp.float32)
            for q in range((CNTP // NS) // 16):
                zcnt[pl.ds(q * 16, 16)] = jnp.zeros((16,), jnp.float32)
            pltpu.sync_copy(zcnt, cacc.at[pl.ds(sid * (CNTP // NS), CNTP // NS)])

        plsc.subcore_barrier()

        def start(k, p):
            si, di, nv, ro, se = bufs[p]
            base = wid * EPW + k * CH
            pltpu.sync_copy(src_hbm.at[pl.ds(base, CH)], si)
            pltpu.sync_copy(dst_hbm.at[pl.ds(base, CH)], di)
            pltpu.sync_copy(nrm_hbm.at[pl.ds(base, CH)], nv)
            pltpu.async_copy(x_hbm.at[si], ro, se)

        def process(p):
            si, di, nv, ro, se = bufs[p]
            pltpu.make_async_copy(x_hbm.at[si], ro, se).wait()
            for g in range(CH // 16):
                nvv = nv[pl.ds(g * 16, 16)]
                for j in range(16):
                    e = g * 16 + j
                    b = lax.gather(
                        nvv, jnp.full((16, 1), j, jnp.int32),
                        lax.GatherDimensionNumbers(
                            offset_dims=(), collapsed_slice_dims=(0,),
                            start_index_map=(0,)),
                        (1,), mode=lax.GatherScatterMode.PROMISE_IN_BOUNDS)
                    for q in range(D // 16):
                        sl = pl.ds(q * 16, 16)
                        ro[e, sl] = ro[e, sl] * b
            pltpu.sync_copy(ro, acc.at[di], add=True)
            if with_counts:
                pltpu.sync_copy(ones, cacc.at[di], add=True)

        start(0, 0)

        @pl.loop(0, NCHUNK, step=2)
        def _chunk(k):
            @pl.when(k + 1 < NCHUNK)
            def _():
                start(k + 1, 1)

            process(0)

            @pl.when(k + 2 < NCHUNK)
            def _():
                start(k + 2, 0)

            @pl.when(k + 1 < NCHUNK)
            def _():
                process(1)

        plsc.subcore_barrier()

        for j in range(-(-NZCOPY // NS)):
            t = sid + j * NS

            @pl.when(t < NZCOPY)
            def _():
                sl = pl.ds(t * ZR, ZR)
                pltpu.sync_copy(acc.at[sl], acc_out.at[cid, sl])
        if with_counts:
            sl = pl.ds(sid * (CNTP // NS), CNTP // NS)
            pltpu.sync_copy(cacc.at[sl], cnt_out.at[cid, sl])

    return pl.kernel(body, out_type=out_type, mesh=mesh, scratch_types=scratch)


_prop_cnt = _make_prop(True)
_prop = _make_prop(False)


def kernel(x, edge_index, norm, y, W_conv, b_conv, Wc1, bc1, Wc2, bc2,
           bn_gamma, bn_beta):
    src = edge_index[0]
    dst = _prep_dst(edge_index[1])

    def half(x_in, gather_i, scatter_i, Wl, bl, cnt=None, pre=None,
             g=None, be=None):
        xe = _enc(x_in, Wl[0], bl[0], Wl[1], bl[1], pre=pre, gamma=g, beta=be)
        if cnt is None:
            acc, cnt = _prop_cnt(xe, gather_i, scatter_i, norm)
        else:
            (acc,) = _prop(xe, gather_i, scatter_i, norm)
        xd, cs, ss = _dec(acc, cnt, Wl[2], bl[2], Wl[3], bl[3])
        return _center(xd, cs, ss), cnt

    c1, cnt_d = half(x, src, dst, W_conv[0], b_conv[0])
    c3, cnt_s = half(c1, dst, src, W_conv[3], b_conv[3],
                     pre="bn", g=bn_gamma[0], be=bn_beta[0])
    c5, _ = half(c3, src, dst, W_conv[1], b_conv[1], cnt=cnt_d,
                 pre="bn", g=bn_gamma[2], be=bn_beta[2])
    c7, _ = half(c5, dst, src, W_conv[4], b_conv[4], cnt=cnt_s,
                 pre="bn", g=bn_gamma[1], be=bn_beta[1])
    c9, _ = half(c7, src, dst, W_conv[2], b_conv[2], cnt=cnt_d,
                 pre="bn", g=bn_gamma[3], be=bn_beta[3])

    edge_score = _classifier(c1, c5, c9, Wc1, bc1, Wc2, bc2)
    return (edge_score, c9, c7)
